# baseline (speedup 1.0000x reference)
import functools

import jax
import jax.numpy as jnp
from jax import lax
from jax.experimental import pallas as pl
from jax.experimental.pallas import tpu as pltpu
from jax.experimental.pallas import tpu_sc as plsc

_B = 1024
_V = 100000

_info = plsc.get_sparse_core_info()
_NC, _NS, _L = _info.num_cores, _info.num_subcores, _info.num_lanes
_NW = _NC * _NS
_BPW = _B // _NW

_mesh = plsc.VectorSubcoreMesh(core_axis_name="c", subcore_axis_name="s")


@functools.partial(
    pl.kernel,
    mesh=_mesh,
    out_type=jax.ShapeDtypeStruct((_B,), jnp.float32),
    scratch_types=[
        pltpu.VMEM((_BPW,), jnp.int32),
        pltpu.VMEM((_BPW, 8, 128), jnp.float32),
        pltpu.VMEM((_BPW,), jnp.float32),
        pltpu.SemaphoreType.DMA,
    ],
    compiler_params=pltpu.CompilerParams(
        needs_layout_passes=False, skip_device_barrier=True
    ),
)
def _sc_gather(prd_hbm, trg_hbm, out_hbm, idx_v, blk_v, res_v, sem):
    wid = lax.axis_index("s") * _NC + lax.axis_index("c")
    base = wid * _BPW
    pltpu.sync_copy(trg_hbm.at[pl.ds(base, _BPW)], idx_v)
    chunks = [idx_v[pl.ds(c * _L, _L)] for c in range(_BPW // _L)]
    copies = []
    for j in range(_BPW):
        t = chunks[j // _L][j % _L]
        col = pl.multiple_of(t & ~127, 128)
        row8 = pl.multiple_of(base + (j & ~7), 8)
        copies.append(
            pltpu.async_copy(
                prd_hbm.at[pl.ds(row8, 8), pl.ds(col, 128)], blk_v.at[j], sem
            )
        )
    for c in copies:
        c.wait()
    for c in range(_BPW // _L):
        rows = c * _L + lax.broadcasted_iota(jnp.int32, (_L,), 0)
        subs = rows & 7
        lanes = chunks[c] & 127
        res_v[pl.ds(c * _L, _L)] = plsc.load_gather(blk_v, [rows, subs, lanes])
    pltpu.sync_copy(res_v, out_hbm.at[pl.ds(base, _BPW)])


def _logsum_body(x_ref, o_ref):
    o_ref[0, 0] = -jnp.sum(jnp.log(x_ref[...]))


@functools.partial(
    pl.kernel,
    mesh=_mesh,
    out_type=jax.ShapeDtypeStruct((_B,), jnp.float32),
    scratch_types=[
        pltpu.VMEM((_BPW,), jnp.float32),
        pltpu.SemaphoreType.DMA,
    ],
    compiler_params=pltpu.CompilerParams(
        needs_layout_passes=False, skip_device_barrier=True
    ),
)
def _sc_trivial(p_hbm, x_hbm, out_hbm, v, sem):
    wid = lax.axis_index("s") * _NC + lax.axis_index("c")
    base = wid * _BPW
    pltpu.sync_copy(x_hbm.at[pl.ds(base, _BPW)], v)
    pltpu.sync_copy(v, out_hbm.at[pl.ds(base, _BPW)])


def kernel(prd, trg):
    vals = _sc_trivial(prd, trg.astype(jnp.float32))
    return vals[0]
    loss = pl.pallas_call(
        _logsum_body,
        out_shape=jax.ShapeDtypeStruct((1, 1), jnp.float32),
        out_specs=pl.BlockSpec(memory_space=pltpu.SMEM),
    )(vals.reshape(8, 128))
    return loss[0, 0]


# speedup vs baseline: 15.8546x; 15.8546x over previous
import functools

import jax
import jax.numpy as jnp
from jax import lax
from jax.experimental import pallas as pl
from jax.experimental.pallas import tpu as pltpu
from jax.experimental.pallas import tpu_sc as plsc

_B = 1024
_V = 100000

_info = plsc.get_sparse_core_info()
_NC, _NS, _L = _info.num_cores, _info.num_subcores, _info.num_lanes
_NW = _NC * _NS
_BPW = _B // _NW

_mesh = plsc.VectorSubcoreMesh(core_axis_name="c", subcore_axis_name="s")


@functools.partial(
    pl.kernel,
    mesh=_mesh,
    out_type=jax.ShapeDtypeStruct((_B,), jnp.float32),
    scratch_types=[
        pltpu.VMEM((_BPW,), jnp.int32),
        pltpu.VMEM((_BPW, _B), jnp.float32),
        pltpu.VMEM((_BPW,), jnp.float32),
        pltpu.SemaphoreType.DMA,
    ],
    compiler_params=pltpu.CompilerParams(
        needs_layout_passes=False, skip_device_barrier=True
    ),
)
def _sc_gather(prdt_hbm, trg_hbm, out_hbm, idx_v, rows_v, res_v, sem):
    wid = lax.axis_index("s") * _NC + lax.axis_index("c")
    base = wid * _BPW
    pltpu.sync_copy(trg_hbm.at[pl.ds(base, _BPW)], idx_v)
    pltpu.async_copy(prdt_hbm.at[idx_v], rows_v, sem).wait()
    for c in range(_BPW // _L):
        rows = c * _L + lax.broadcasted_iota(jnp.int32, (_L,), 0)
        cols = base + rows
        res_v[pl.ds(c * _L, _L)] = plsc.load_gather(rows_v, [rows, cols])
    pltpu.sync_copy(res_v, out_hbm.at[pl.ds(base, _BPW)])


def _logsum_body(x_ref, o_ref):
    o_ref[0, 0] = -jnp.sum(jnp.log(x_ref[...]))


def kernel(prd, trg):
    vals = _sc_gather(prd.T, trg.astype(jnp.int32))
    loss = pl.pallas_call(
        _logsum_body,
        out_shape=jax.ShapeDtypeStruct((1, 1), jnp.float32),
        out_specs=pl.BlockSpec(memory_space=pltpu.SMEM),
    )(vals.reshape(8, 128))
    return loss[0, 0]


# speedup vs baseline: 16.7823x; 1.0585x over previous
import functools

import jax
import jax.numpy as jnp
from jax import lax
from jax.experimental import pallas as pl
from jax.experimental.pallas import tpu as pltpu
from jax.experimental.pallas import tpu_sc as plsc

_B = 1024
_V = 100000

_info = plsc.get_sparse_core_info()
_L = _info.num_lanes
_NW = 16
_BPW = _B // _NW

_LN2 = 0.6931471805599453

_mesh = plsc.VectorSubcoreMesh(
    core_axis_name="c", subcore_axis_name="s", num_cores=1
)


def _neg_log(v):
    bits = lax.bitcast_convert_type(v, jnp.int32)
    e = lax.convert_element_type(
        lax.shift_right_logical(bits, 23) - 127, jnp.float32
    )
    m = lax.bitcast_convert_type(
        (bits & 0x7FFFFF) | 0x3F800000, jnp.float32
    )
    z = (m - 1.0) / (m + 1.0)
    z2 = z * z
    p = 1.0 + z2 * (
        1.0 / 3.0 + z2 * (1.0 / 5.0 + z2 * (1.0 / 7.0 + z2 * (1.0 / 9.0)))
    )
    return -(e * _LN2 + 2.0 * z * p)


@functools.partial(
    pl.kernel,
    mesh=_mesh,
    out_type=jax.ShapeDtypeStruct((_L,), jnp.float32),
    scratch_types=[
        pltpu.VMEM((_BPW,), jnp.int32),
        pltpu.VMEM((_BPW, _B), jnp.float32),
        pltpu.VMEM((_L,), jnp.float32),
        pltpu.VMEM((_L,), jnp.float32),
        pltpu.VMEM_SHARED((_L,), jnp.float32),
        pltpu.SemaphoreType.DMA,
    ],
    compiler_params=pltpu.CompilerParams(
        needs_layout_passes=False, skip_device_barrier=True
    ),
)
def _sc_loss(prdt_hbm, trg_hbm, out_hbm, idx_v, rows_v, acc_v, red_v,
             shared, sem):
    wid = lax.axis_index("s")
    base = wid * _BPW

    @pl.when(wid == 0)
    def _zero():
        acc_v[...] = jnp.zeros((_L,), jnp.float32)
        pltpu.sync_copy(acc_v, shared)

    plsc.subcore_barrier()
    pltpu.sync_copy(trg_hbm.at[pl.ds(base, _BPW)], idx_v)
    pltpu.async_copy(prdt_hbm.at[idx_v], rows_v, sem).wait()
    acc = jnp.zeros((_L,), jnp.float32)
    for c in range(_BPW // _L):
        rows = c * _L + lax.broadcasted_iota(jnp.int32, (_L,), 0)
        cols = base + rows
        acc = acc + _neg_log(plsc.load_gather(rows_v, [rows, cols]))
    acc_v[...] = acc
    lanes = lax.broadcasted_iota(jnp.int32, (_L,), 0)
    pltpu.sync_copy(acc_v, shared.at[lanes], add=True)
    plsc.subcore_barrier()

    @pl.when(wid == 0)
    def _reduce():
        pltpu.sync_copy(shared, red_v)
        total = jnp.sum(red_v[...])
        red_v[...] = jnp.full((_L,), total, jnp.float32)
        pltpu.sync_copy(red_v, out_hbm)


def kernel(prd, trg):
    vals = _sc_loss(prd.T, trg.astype(jnp.int32))
    return vals[0]
